# Initial kernel scaffold; baseline (speedup 1.0000x reference)
import jax
import jax.numpy as jnp
from jax.experimental import pallas as pl


def kernel(x, emb_word, emb_ngram2, emb_ngram3, W1, b1, W2, b2):
    raise NotImplementedError("write your pallas kernel here")


# speedup vs baseline: 5.8198x; 5.8198x over previous
import functools

import jax
import jax.numpy as jnp
from jax import lax
from jax.experimental import pallas as pl
from jax.experimental.pallas import tpu as pltpu
from jax.experimental.pallas import tpu_sc as plsc

B = 16384
L = 20
D = 128
TD = 3 * D
H = 1024
OUT = 1024

NC = 2
NS = 16
NW = NC * NS

SPW = B // NW
CHUNK = 32
NCH = SPW // CHUNK
CL = CHUNK * L
GW = 128
NSUB = CL // GW

_MESH = plsc.VectorSubcoreMesh(core_axis_name="c", subcore_axis_name="s")


@functools.partial(
    pl.kernel,
    mesh=_MESH,
    out_type=jax.ShapeDtypeStruct((B, TD), jnp.float32),
    scratch_types=[
        pltpu.VMEM((NSUB, GW), jnp.int32),
        pltpu.VMEM((CL, D), jnp.float32),
        pltpu.VMEM((CHUNK, TD), jnp.float32),
        pltpu.SemaphoreType.DMA,
    ],
)
def _pool(xw, x2, x3, tw, t2, t3, out, idx_v, rows_v, acc_v, sem):
    wid = lax.axis_index("s") * NC + lax.axis_index("c")
    base = wid * SPW

    def chunk_body(c, carry):
        s0 = base + c * CHUNK
        i0 = s0 * L

        for t, (xh, th) in enumerate(((xw, tw), (x2, t2), (x3, t3))):
            for j in range(NSUB):
                pltpu.sync_copy(xh.at[pl.ds(i0 + j * GW, GW)], idx_v.at[j])
            cps = [
                pltpu.async_copy(th.at[idx_v.at[j]],
                                 rows_v.at[pl.ds(j * GW, GW)], sem)
                for j in range(NSUB)
            ]
            for cp in cps:
                cp.wait()

            def samp_body(s, carry2):
                r = s * L
                for v in range(D // 16):
                    col = pl.ds(v * 16, 16)
                    accv = rows_v[r, col]
                    for l in range(1, L):
                        accv = accv + rows_v[r + l, col]
                    acc_v[s, pl.ds(t * D + v * 16, 16)] = accv * (1.0 / L)
                return carry2

            lax.fori_loop(0, CHUNK, samp_body, 0)

        pltpu.sync_copy(acc_v, out.at[pl.ds(s0, CHUNK)])
        return carry

    lax.fori_loop(0, NCH, chunk_body, 0)


BM = 512


def _mlp_body(p_ref, w1_ref, b1_ref, w2_ref, b2_ref, o_ref):
    h = jnp.dot(p_ref[...], w1_ref[...], preferred_element_type=jnp.float32)
    h = jnp.maximum(h + b1_ref[...], 0.0)
    o_ref[...] = (
        jnp.dot(h, w2_ref[...], preferred_element_type=jnp.float32)
        + b2_ref[...]
    )


_mlp = pl.pallas_call(
    _mlp_body,
    grid=(B // BM,),
    in_specs=[
        pl.BlockSpec((BM, TD), lambda i: (i, 0)),
        pl.BlockSpec((TD, H), lambda i: (0, 0)),
        pl.BlockSpec((1, H), lambda i: (0, 0)),
        pl.BlockSpec((H, OUT), lambda i: (0, 0)),
        pl.BlockSpec((1, OUT), lambda i: (0, 0)),
    ],
    out_specs=pl.BlockSpec((BM, OUT), lambda i: (i, 0)),
    out_shape=jax.ShapeDtypeStruct((B, OUT), jnp.float32),
)


def kernel(x, emb_word, emb_ngram2, emb_ngram3, W1, b1, W2, b2):
    xw = x[0].reshape(B * L)
    xb = x[2].reshape(B * L)
    xt = x[3].reshape(B * L)
    pooled = _pool(xw, xb, xt, emb_word, emb_ngram2, emb_ngram3)
    return _mlp(pooled, W1, b1.reshape(1, H), W2, b2.reshape(1, OUT))


# speedup vs baseline: 6.8355x; 1.1745x over previous
import functools

import jax
import jax.numpy as jnp
from jax import lax
from jax.experimental import pallas as pl
from jax.experimental.pallas import tpu as pltpu
from jax.experimental.pallas import tpu_sc as plsc

B = 16384
L = 20
D = 128
TD = 3 * D
H = 1024
OUT = 1024

NC = 2
NS = 16
NW = NC * NS

SPW = B // NW
CHUNK = 32
NCH = SPW // CHUNK
CL = CHUNK * L
GW = 128
NSUB = CL // GW
IR = SPW * L // GW

_UPTO = [((j + 1) * GW) // L for j in range(NSUB - 1)] + [CHUNK]

_MESH = plsc.VectorSubcoreMesh(core_axis_name="c", subcore_axis_name="s")


@functools.partial(
    pl.kernel,
    mesh=_MESH,
    out_type=jax.ShapeDtypeStruct((B, TD), jnp.float32),
    scratch_types=[
        pltpu.VMEM((3 * IR, GW), jnp.int32),
        pltpu.VMEM((CL, D), jnp.float32),
        pltpu.VMEM((CHUNK, TD), jnp.float32),
        pltpu.SemaphoreType.DMA((NSUB,)),
    ],
)
def _pool(xw, x2, x3, tw, t2, t3, out, idx_v, rows_v, acc_v, sems):
    wid = lax.axis_index("s") * NC + lax.axis_index("c")
    base = wid * SPW

    for t, xh in enumerate((xw, x2, x3)):
        pltpu.sync_copy(xh.at[pl.ds(wid * IR, IR)],
                        idx_v.at[pl.ds(t * IR, IR)])

    def chunk_body(c, carry):
        s0 = base + c * CHUNK

        for t, th in enumerate((tw, t2, t3)):
            for j in range(NSUB):
                pltpu.async_copy(th.at[idx_v.at[t * IR + c * NSUB + j]],
                                 rows_v.at[pl.ds(j * GW, GW)], sems.at[j])

            def samp_body(s, carry2):
                r = s * L
                for v in range(D // 16):
                    col = pl.ds(v * 16, 16)
                    accv = rows_v[r, col]
                    for l in range(1, L):
                        accv = accv + rows_v[r + l, col]
                    acc_v[s, pl.ds(t * D + v * 16, 16)] = accv * (1.0 / L)
                return carry2

            def drain_body(j, carry2):
                pltpu.make_async_copy(th.at[pl.ds(0, GW)],
                                      rows_v.at[pl.ds(0, GW)],
                                      sems.at[j]).wait()
                lo = j * GW // L
                hi = jnp.minimum((j + 1) * GW // L, CHUNK)
                lax.fori_loop(lo, hi, samp_body, 0)
                return carry2

            lax.fori_loop(0, NSUB, drain_body, 0)

        pltpu.sync_copy(acc_v, out.at[pl.ds(s0, CHUNK)])
        return carry

    lax.fori_loop(0, NCH, chunk_body, 0)


BM = 512


def _mlp_body(p_ref, w1_ref, b1_ref, w2_ref, b2_ref, o_ref):
    h = jnp.dot(p_ref[...], w1_ref[...], preferred_element_type=jnp.float32)
    h = jnp.maximum(h + b1_ref[...], 0.0)
    o_ref[...] = (
        jnp.dot(h, w2_ref[...], preferred_element_type=jnp.float32)
        + b2_ref[...]
    )


_mlp = pl.pallas_call(
    _mlp_body,
    grid=(B // BM,),
    in_specs=[
        pl.BlockSpec((BM, TD), lambda i: (i, 0)),
        pl.BlockSpec((TD, H), lambda i: (0, 0)),
        pl.BlockSpec((1, H), lambda i: (0, 0)),
        pl.BlockSpec((H, OUT), lambda i: (0, 0)),
        pl.BlockSpec((1, OUT), lambda i: (0, 0)),
    ],
    out_specs=pl.BlockSpec((BM, OUT), lambda i: (i, 0)),
    out_shape=jax.ShapeDtypeStruct((B, OUT), jnp.float32),
)


def kernel(x, emb_word, emb_ngram2, emb_ngram3, W1, b1, W2, b2):
    xw = x[0].reshape(B * L // GW, GW)
    xb = x[2].reshape(B * L // GW, GW)
    xt = x[3].reshape(B * L // GW, GW)
    pooled = _pool(xw, xb, xt, emb_word, emb_ngram2, emb_ngram3)
    return _mlp(pooled, W1, b1.reshape(1, H), W2, b2.reshape(1, OUT))
